# Initial kernel scaffold; baseline (speedup 1.0000x reference)
import jax
import jax.numpy as jnp
from jax.experimental import pallas as pl


def kernel(x):
    raise NotImplementedError("write your pallas kernel here")


# speedup vs baseline: 1.4311x; 1.4311x over previous
import functools

import jax
import jax.numpy as jnp
from jax import lax
from jax.experimental import pallas as pl
from jax.experimental.pallas import tpu as pltpu
from jax.experimental.pallas import tpu_sc as plsc

B, L, D = 4096, 200, 128
NC, NS, LANES = 2, 16, 16
NW = NC * NS
BPW = B // NW
NG = (L + LANES - 1) // LANES
LP = NG * LANES
HALF = 112
EPS = 1e-6


def _body(x_hbm, out_hbm, xb, buf2, g_flat, g2, sem):
    wid = lax.axis_index("s") * NC + lax.axis_index("c")

    def one_batch(i, _):
        b = wid * BPW + i
        base = b * L
        pltpu.sync_copy(x_hbm.at[pl.ds(base, L)], xb)

        def one_group(g, carry):
            last_valid, nm_vec = carry
            t = g * LANES + lax.iota(jnp.int32, LANES)
            tc = jnp.minimum(t, L - 1)
            acc = jnp.zeros((LANES,), jnp.float32)
            for d in range(D):
                v = plsc.load_gather(xb, [tc, jnp.full((LANES,), d, jnp.int32)])
                acc = jnp.maximum(acc, jnp.abs(v))
            mask = acc <= EPS
            oob = t > (L - 1)
            val = jnp.where(mask | oob, 0, t)
            f_vec = jnp.maximum(plsc.cummax(val),
                                jnp.full((LANES,), last_valid, jnp.int32))
            nm_vec = nm_vec + plsc.all_reduce_population_count(mask & (~oob))
            g_flat[pl.ds(g * LANES, LANES)] = f_vec
            return jnp.max(f_vec), nm_vec

        _, nm_vec = lax.fori_loop(
            0, NG, one_group,
            (jnp.int32(0), jnp.zeros((LANES,), jnp.int32)))
        nm_s = jnp.max(nm_vec)

        @pl.when(nm_s == 0)
        def _common():
            pltpu.sync_copy(xb, out_hbm.at[pl.ds(base, L)])

        @pl.when(nm_s > 0)
        def _rare():
            base_vec = jnp.full((LANES,), base, jnp.int32)
            for row in range(2):
                for j in range(HALF // LANES):
                    off = row * HALF + j * LANES
                    if off < LP:
                        vec = g_flat[pl.ds(off, LANES)] + base_vec
                    else:
                        vec = base_vec
                    g2[row, pl.ds(j * LANES, LANES)] = vec
            pltpu.async_copy(
                x_hbm.at[g2.at[0]], buf2.at[pl.ds(0, HALF)], sem).wait()
            pltpu.async_copy(
                x_hbm.at[g2.at[1]], buf2.at[pl.ds(HALF, HALF)], sem).wait()
            pltpu.sync_copy(buf2.at[pl.ds(0, L)], out_hbm.at[pl.ds(base, L)])

        return 0

    lax.fori_loop(0, BPW, one_batch, 0)


@jax.jit
def _imputer(xf):
    mesh = plsc.VectorSubcoreMesh(core_axis_name="c", subcore_axis_name="s")
    return pl.kernel(
        _body,
        out_type=jax.ShapeDtypeStruct((B * L, D), jnp.float32),
        mesh=mesh,
        compiler_params=pltpu.CompilerParams(needs_layout_passes=False),
        scratch_types=[
            pltpu.VMEM((L, D), jnp.float32),
            pltpu.VMEM((2 * HALF, D), jnp.float32),
            pltpu.VMEM((LP,), jnp.int32),
            pltpu.VMEM((2, HALF), jnp.int32),
            pltpu.SemaphoreType.DMA,
        ],
    )(xf)


def kernel(x):
    batch_dims = x.shape[:-2]
    xf = x.reshape(B * L, D)
    return _imputer(xf).reshape(*batch_dims, L, D)


# speedup vs baseline: 4.3872x; 3.0657x over previous
import jax
import jax.numpy as jnp
import numpy as np
from jax import lax
from jax.experimental import pallas as pl
from jax.experimental.pallas import tpu as pltpu
from jax.experimental.pallas import tpu_sc as plsc

B, L, D = 4096, 200, 128
NC, NS, LANES = 2, 16, 16
NW = NC * NS
BPW = B // NW
NG = (L + LANES - 1) // LANES
LP = NG * LANES
HALF = 112
DK = D // LANES
SIGN_OFF = 0x7FFFFFFF
EPS_BITS = int(np.float32(1e-6).view(np.int32))


def _row_masks(xb, g):
    mv = jnp.zeros((LANES,), jnp.int32)
    iota = lax.iota(jnp.int32, LANES)
    for tl in range(LANES):
        t = g * LANES + tl
        acc = jnp.zeros((LANES,), jnp.int32)
        for k in range(DK):
            v = xb[t, pl.ds(k * LANES, LANES)]
            vi = plsc.bitcast(v, jnp.int32) & SIGN_OFF
            acc = jnp.maximum(acc, vi)
        lanemask = acc <= EPS_BITS
        pc = plsc.all_reduce_population_count(lanemask)
        rowm = pc == LANES
        mv = mv | jnp.where(rowm & (iota == tl), 1, 0)
    return mv


def _body(x_hbm, out_hbm, xb, g_flat, g2, sem):
    wid = lax.axis_index("s") * NC + lax.axis_index("c")

    def one_batch(i, _):
        b = wid * BPW + i
        base = b * L
        pltpu.sync_copy(x_hbm.at[pl.ds(base, L)], xb.at[pl.ds(0, L)])

        def one_group(g, carry):
            last_valid, nm_vec = carry
            mv = _row_masks(xb, g)
            t = g * LANES + lax.iota(jnp.int32, LANES)
            oob = t > (L - 1)
            masked = (mv == 1) & (~oob)
            val = jnp.where(masked | oob, 0, t)
            f_vec = jnp.maximum(plsc.cummax(val),
                                jnp.full((LANES,), last_valid, jnp.int32))
            nm_vec = nm_vec + plsc.all_reduce_population_count(masked)
            g_flat[pl.ds(g * LANES, LANES)] = f_vec
            return jnp.max(f_vec), nm_vec

        _, nm_vec = lax.fori_loop(
            0, NG, one_group,
            (jnp.int32(0), jnp.zeros((LANES,), jnp.int32)))
        nm_s = jnp.max(nm_vec)

        @pl.when(nm_s > 0)
        def _rare():
            base_vec = jnp.full((LANES,), base, jnp.int32)
            for row in range(2):
                for j in range(HALF // LANES):
                    off = row * HALF + j * LANES
                    if off < LP:
                        vec = g_flat[pl.ds(off, LANES)] + base_vec
                    else:
                        vec = base_vec
                    g2[row, pl.ds(j * LANES, LANES)] = vec
            pltpu.async_copy(
                x_hbm.at[g2.at[0]], xb.at[pl.ds(0, HALF)], sem).wait()
            pltpu.async_copy(
                x_hbm.at[g2.at[1]], xb.at[pl.ds(HALF, HALF)], sem).wait()

        pltpu.sync_copy(xb.at[pl.ds(0, L)], out_hbm.at[pl.ds(base, L)])
        return 0

    lax.fori_loop(0, BPW, one_batch, 0)


@jax.jit
def _imputer(xf):
    mesh = plsc.VectorSubcoreMesh(core_axis_name="c", subcore_axis_name="s")
    return pl.kernel(
        _body,
        out_type=jax.ShapeDtypeStruct((B * L, D), jnp.float32),
        mesh=mesh,
        compiler_params=pltpu.CompilerParams(needs_layout_passes=False),
        scratch_types=[
            pltpu.VMEM((2 * HALF, D), jnp.float32),
            pltpu.VMEM((LP,), jnp.int32),
            pltpu.VMEM((2, HALF), jnp.int32),
            pltpu.SemaphoreType.DMA,
        ],
    )(xf)


def kernel(x):
    batch_dims = x.shape[:-2]
    xf = x.reshape(B * L, D)
    return _imputer(xf).reshape(*batch_dims, L, D)


# speedup vs baseline: 7.8707x; 1.7940x over previous
import jax
import jax.numpy as jnp
import numpy as np
from jax import lax
from jax.experimental import pallas as pl
from jax.experimental.pallas import tpu as pltpu
from jax.experimental.pallas import tpu_sc as plsc

B, L, D = 4096, 200, 128
NC, NS, LANES = 2, 16, 16
NW = NC * NS
BPW = B // NW
NG = (L + LANES - 1) // LANES
LP = NG * LANES
HALF = 112
DK = D // LANES
SIGN_OFF = 0x7FFFFFFF
EPS_BITS = int(np.float32(1e-6).view(np.int32))


def _row_masks(xb, g):
    mv = jnp.zeros((LANES,), jnp.int32)
    iota = lax.iota(jnp.int32, LANES)
    for tl in range(LANES):
        t = g * LANES + tl
        acc = jnp.zeros((LANES,), jnp.int32)
        for k in range(DK):
            v = xb[t, pl.ds(k * LANES, LANES)]
            vi = plsc.bitcast(v, jnp.int32) & SIGN_OFF
            acc = jnp.maximum(acc, vi)
        lanemask = acc <= EPS_BITS
        pc = plsc.all_reduce_population_count(lanemask)
        rowm = pc == LANES
        mv = mv | jnp.where(rowm & (iota == tl), 1, 0)
    return mv


NBUF = 4


def _body(x_hbm, out_hbm, xb0, xb1, xb2, xb3, g_flat, g2,
          si0, si1, si2, si3, so0, so1, so2, so3, gsem):
    wid = lax.axis_index("s") * NC + lax.axis_index("c")
    bufs = (xb0, xb1, xb2, xb3)
    sis = (si0, si1, si2, si3)
    sos = (so0, so1, so2, so3)

    def in_copy(j, k):
        base = (wid * BPW + j) * L
        return pltpu.make_async_copy(
            x_hbm.at[pl.ds(base, L)], bufs[k].at[pl.ds(0, L)], sis[k])

    def out_copy(j, k):
        base = (wid * BPW + j) * L
        return pltpu.make_async_copy(
            bufs[k].at[pl.ds(0, L)], out_hbm.at[pl.ds(base, L)], sos[k])

    in_copy(0, 0).start()

    def quad(i, _):
        for k in range(NBUF):
            j = NBUF * i + k
            b = wid * BPW + j
            base = b * L

            kn = (k + 1) % NBUF
            if k < NBUF - 1:
                @pl.when(i >= 1)
                def _drain():
                    out_copy(j - (NBUF - 1), kn).wait()
            else:
                out_copy(j - (NBUF - 1), kn).wait()

            if k < NBUF - 1:
                in_copy(j + 1, kn).start()
            else:
                @pl.when(i < BPW // NBUF - 1)
                def _nxt():
                    in_copy(j + 1, kn).start()

            in_copy(j, k).wait()
            xb = bufs[k]

            def one_group(g, carry):
                last_valid, nm_vec = carry
                mv = _row_masks(xb, g)
                t = g * LANES + lax.iota(jnp.int32, LANES)
                oob = t > (L - 1)
                masked = (mv == 1) & (~oob)
                val = jnp.where(masked | oob, 0, t)
                f_vec = jnp.maximum(plsc.cummax(val),
                                    jnp.full((LANES,), last_valid, jnp.int32))
                nm_vec = nm_vec + plsc.all_reduce_population_count(masked)
                g_flat[pl.ds(g * LANES, LANES)] = f_vec
                return jnp.max(f_vec), nm_vec

            _, nm_vec = lax.fori_loop(
                0, NG, one_group,
                (jnp.int32(0), jnp.zeros((LANES,), jnp.int32)))
            nm_s = jnp.max(nm_vec)

            @pl.when(nm_s > 0)
            def _rare():
                base_vec = jnp.full((LANES,), base, jnp.int32)
                for row in range(2):
                    for jj in range(HALF // LANES):
                        off = row * HALF + jj * LANES
                        if off < LP:
                            vec = g_flat[pl.ds(off, LANES)] + base_vec
                        else:
                            vec = base_vec
                        g2[row, pl.ds(jj * LANES, LANES)] = vec
                pltpu.async_copy(
                    x_hbm.at[g2.at[0]], xb.at[pl.ds(0, HALF)], gsem).wait()
                pltpu.async_copy(
                    x_hbm.at[g2.at[1]], xb.at[pl.ds(HALF, HALF)], gsem).wait()

            out_copy(j, k).start()
        return 0

    lax.fori_loop(0, BPW // NBUF, quad, 0)
    for j in range(BPW - (NBUF - 1), BPW):
        out_copy(j, j % NBUF).wait()


@jax.jit
def _imputer(xf):
    mesh = plsc.VectorSubcoreMesh(core_axis_name="c", subcore_axis_name="s")
    return pl.kernel(
        _body,
        out_type=jax.ShapeDtypeStruct((B * L, D), jnp.float32),
        mesh=mesh,
        compiler_params=pltpu.CompilerParams(needs_layout_passes=False),
        scratch_types=(
            [pltpu.VMEM((2 * HALF, D), jnp.float32) for _ in range(NBUF)]
            + [pltpu.VMEM((LP,), jnp.int32),
               pltpu.VMEM((2, HALF), jnp.int32)]
            + [pltpu.SemaphoreType.DMA for _ in range(2 * NBUF + 1)]
        ),
    )(xf)


def kernel(x):
    batch_dims = x.shape[:-2]
    xf = x.reshape(B * L, D)
    return _imputer(xf).reshape(*batch_dims, L, D)


# speedup vs baseline: 7.9829x; 1.0143x over previous
import jax
import jax.numpy as jnp
import numpy as np
from jax import lax
from jax.experimental import pallas as pl
from jax.experimental.pallas import tpu as pltpu
from jax.experimental.pallas import tpu_sc as plsc

B, L, D = 4096, 200, 128
NC, NS, LANES = 2, 16, 16
NW = NC * NS
BPW = B // NW
NG = (L + LANES - 1) // LANES
LP = NG * LANES
HALF = 112
DK = D // LANES
SIGN_OFF = 0x7FFFFFFF
EPS_BITS = int(np.float32(1e-6).view(np.int32))


def _row_masks(xb, g):
    mv = jnp.zeros((LANES,), jnp.int32)
    iota = lax.iota(jnp.int32, LANES)
    for tl in range(LANES):
        t = g * LANES + tl
        acc = jnp.zeros((LANES,), jnp.int32)
        for k in range(DK):
            v = xb[t, pl.ds(k * LANES, LANES)]
            vi = plsc.bitcast(v, jnp.int32) & SIGN_OFF
            acc = jnp.maximum(acc, vi)
        lanemask = acc <= EPS_BITS
        pc = plsc.all_reduce_population_count(lanemask)
        rowm = pc == LANES
        mv = mv | jnp.where(rowm & (iota == tl), 1, 0)
    return mv


NBUF = 4


def _body(x_hbm, out_hbm, xb0, xb1, xb2, xb3, g_flat, g2,
          si0, si1, si2, si3, so0, so1, so2, so3, gsem):
    wid = lax.axis_index("s") * NC + lax.axis_index("c")
    bufs = (xb0, xb1, xb2, xb3)
    sis = (si0, si1, si2, si3)
    sos = (so0, so1, so2, so3)

    def in_copy(j, k):
        base = (wid * BPW + j) * L
        return pltpu.make_async_copy(
            x_hbm.at[pl.ds(base, L)], bufs[k].at[pl.ds(0, L)], sis[k])

    def out_copy(j, k):
        base = (wid * BPW + j) * L
        return pltpu.make_async_copy(
            bufs[k].at[pl.ds(0, L)], out_hbm.at[pl.ds(base, L)], sos[k])

    in_copy(0, 0).start()

    def quad(i, _):
        for k in range(NBUF):
            j = NBUF * i + k
            b = wid * BPW + j
            base = b * L

            kn = (k + 1) % NBUF
            if k < NBUF - 1:
                @pl.when(i >= 1)
                def _drain():
                    out_copy(j - (NBUF - 1), kn).wait()
            else:
                out_copy(j - (NBUF - 1), kn).wait()

            if k < NBUF - 1:
                in_copy(j + 1, kn).start()
            else:
                @pl.when(i < BPW // NBUF - 1)
                def _nxt():
                    in_copy(j + 1, kn).start()

            in_copy(j, k).wait()
            xb = bufs[k]

            zeros16 = jnp.zeros((LANES,), jnp.int32)
            sus = jnp.zeros((LANES,), jnp.int32)
            for g in range(NG):
                tcs = jnp.minimum(g * LANES + lax.iota(jnp.int32, LANES),
                                  L - 1)
                v0 = plsc.load_gather(xb, [tcs, zeros16])
                vi0 = plsc.bitcast(v0, jnp.int32) & SIGN_OFF
                sus = sus | jnp.where(vi0 <= EPS_BITS, 1, 0)
            ns = jnp.max(sus)

            @pl.when(ns > 0)
            def _full():
                def one_group(g, carry):
                    last_valid, nm_vec = carry
                    mv = _row_masks(xb, g)
                    t = g * LANES + lax.iota(jnp.int32, LANES)
                    oob = t > (L - 1)
                    masked = (mv == 1) & (~oob)
                    val = jnp.where(masked | oob, 0, t)
                    f_vec = jnp.maximum(
                        plsc.cummax(val),
                        jnp.full((LANES,), last_valid, jnp.int32))
                    nm_vec = nm_vec + plsc.all_reduce_population_count(masked)
                    g_flat[pl.ds(g * LANES, LANES)] = f_vec
                    return jnp.max(f_vec), nm_vec

                _, nm_vec = lax.fori_loop(
                    0, NG, one_group,
                    (jnp.int32(0), jnp.zeros((LANES,), jnp.int32)))
                nm_s = jnp.max(nm_vec)

                @pl.when(nm_s > 0)
                def _rare():
                    base_vec = jnp.full((LANES,), base, jnp.int32)
                    for row in range(2):
                        for jj in range(HALF // LANES):
                            off = row * HALF + jj * LANES
                            if off < LP:
                                vec = g_flat[pl.ds(off, LANES)] + base_vec
                            else:
                                vec = base_vec
                            g2[row, pl.ds(jj * LANES, LANES)] = vec
                    pltpu.async_copy(
                        x_hbm.at[g2.at[0]], xb.at[pl.ds(0, HALF)],
                        gsem).wait()
                    pltpu.async_copy(
                        x_hbm.at[g2.at[1]], xb.at[pl.ds(HALF, HALF)],
                        gsem).wait()

            out_copy(j, k).start()
        return 0

    lax.fori_loop(0, BPW // NBUF, quad, 0)
    for j in range(BPW - (NBUF - 1), BPW):
        out_copy(j, j % NBUF).wait()


@jax.jit
def _imputer(xf):
    mesh = plsc.VectorSubcoreMesh(core_axis_name="c", subcore_axis_name="s")
    return pl.kernel(
        _body,
        out_type=jax.ShapeDtypeStruct((B * L, D), jnp.float32),
        mesh=mesh,
        compiler_params=pltpu.CompilerParams(needs_layout_passes=False),
        scratch_types=(
            [pltpu.VMEM((2 * HALF, D), jnp.float32) for _ in range(NBUF)]
            + [pltpu.VMEM((LP,), jnp.int32),
               pltpu.VMEM((2, HALF), jnp.int32)]
            + [pltpu.SemaphoreType.DMA for _ in range(2 * NBUF + 1)]
        ),
    )(xf)


def kernel(x):
    batch_dims = x.shape[:-2]
    xf = x.reshape(B * L, D)
    return _imputer(xf).reshape(*batch_dims, L, D)


# speedup vs baseline: 8.0239x; 1.0051x over previous
import jax
import jax.numpy as jnp
import numpy as np
from jax import lax
from jax.experimental import pallas as pl
from jax.experimental.pallas import tpu as pltpu
from jax.experimental.pallas import tpu_sc as plsc

B, L, D = 4096, 200, 128
NC, NS, LANES = 2, 16, 16
NW = NC * NS
BPW = B // NW
NG = (L + LANES - 1) // LANES
LP = NG * LANES
HALF = 112
DK = D // LANES
SIGN_OFF = 0x7FFFFFFF
EPS_BITS = int(np.float32(1e-6).view(np.int32))


def _row_masks(xb, g):
    mv = jnp.zeros((LANES,), jnp.int32)
    iota = lax.iota(jnp.int32, LANES)
    for tl in range(LANES):
        t = g * LANES + tl
        acc = jnp.zeros((LANES,), jnp.int32)
        for k in range(DK):
            v = xb[t, pl.ds(k * LANES, LANES)]
            vi = plsc.bitcast(v, jnp.int32) & SIGN_OFF
            acc = jnp.maximum(acc, vi)
        lanemask = acc <= EPS_BITS
        pc = plsc.all_reduce_population_count(lanemask)
        rowm = pc == LANES
        mv = mv | jnp.where(rowm & (iota == tl), 1, 0)
    return mv


NBUF = 4


def _body(x_hbm, out_hbm, xb0, xb1, xb2, xb3, g_flat, g2,
          si0, si1, si2, si3, so0, so1, so2, so3, gsem):
    wid = lax.axis_index("s") * NC + lax.axis_index("c")
    bufs = (xb0, xb1, xb2, xb3)
    sis = (si0, si1, si2, si3)
    sos = (so0, so1, so2, so3)

    def in_copy(j, k):
        base = (wid * BPW + j) * L
        return pltpu.make_async_copy(
            x_hbm.at[pl.ds(base, L)], bufs[k].at[pl.ds(0, L)], sis[k])

    def out_copy(j, k):
        base = (wid * BPW + j) * L
        return pltpu.make_async_copy(
            bufs[k].at[pl.ds(0, L)], out_hbm.at[pl.ds(base, L)], sos[k])

    in_copy(0, 0).start()
    in_copy(1, 1).start()

    def quad(i, _):
        for k in range(NBUF):
            j = NBUF * i + k
            b = wid * BPW + j
            base = b * L

            kn = (k + 2) % NBUF
            if k < 2:
                @pl.when(i >= 1)
                def _drain():
                    out_copy(j - 2, kn).wait()
            else:
                out_copy(j - 2, kn).wait()

            if k < 2:
                in_copy(j + 2, kn).start()
            else:
                @pl.when(i < BPW // NBUF - 1)
                def _nxt():
                    in_copy(j + 2, kn).start()

            in_copy(j, k).wait()
            xb = bufs[k]

            zeros16 = jnp.zeros((LANES,), jnp.int32)
            sus = jnp.zeros((LANES,), jnp.int32)
            for g in range(NG):
                tcs = jnp.minimum(g * LANES + lax.iota(jnp.int32, LANES),
                                  L - 1)
                v0 = plsc.load_gather(xb, [tcs, zeros16])
                vi0 = plsc.bitcast(v0, jnp.int32) & SIGN_OFF
                sus = sus | jnp.where(vi0 <= EPS_BITS, 1, 0)
            ns = jnp.max(sus)

            @pl.when(ns > 0)
            def _full():
                def one_group(g, carry):
                    last_valid, nm_vec = carry
                    mv = _row_masks(xb, g)
                    t = g * LANES + lax.iota(jnp.int32, LANES)
                    oob = t > (L - 1)
                    masked = (mv == 1) & (~oob)
                    val = jnp.where(masked | oob, 0, t)
                    f_vec = jnp.maximum(
                        plsc.cummax(val),
                        jnp.full((LANES,), last_valid, jnp.int32))
                    nm_vec = nm_vec + plsc.all_reduce_population_count(masked)
                    g_flat[pl.ds(g * LANES, LANES)] = f_vec
                    return jnp.max(f_vec), nm_vec

                _, nm_vec = lax.fori_loop(
                    0, NG, one_group,
                    (jnp.int32(0), jnp.zeros((LANES,), jnp.int32)))
                nm_s = jnp.max(nm_vec)

                @pl.when(nm_s > 0)
                def _rare():
                    base_vec = jnp.full((LANES,), base, jnp.int32)
                    for row in range(2):
                        for jj in range(HALF // LANES):
                            off = row * HALF + jj * LANES
                            if off < LP:
                                vec = g_flat[pl.ds(off, LANES)] + base_vec
                            else:
                                vec = base_vec
                            g2[row, pl.ds(jj * LANES, LANES)] = vec
                    pltpu.async_copy(
                        x_hbm.at[g2.at[0]], xb.at[pl.ds(0, HALF)],
                        gsem).wait()
                    pltpu.async_copy(
                        x_hbm.at[g2.at[1]], xb.at[pl.ds(HALF, HALF)],
                        gsem).wait()

            out_copy(j, k).start()
        return 0

    lax.fori_loop(0, BPW // NBUF, quad, 0)
    for j in range(BPW - 2, BPW):
        out_copy(j, j % NBUF).wait()


@jax.jit
def _imputer(xf):
    mesh = plsc.VectorSubcoreMesh(core_axis_name="c", subcore_axis_name="s")
    return pl.kernel(
        _body,
        out_type=jax.ShapeDtypeStruct((B * L, D), jnp.float32),
        mesh=mesh,
        compiler_params=pltpu.CompilerParams(needs_layout_passes=False),
        scratch_types=(
            [pltpu.VMEM((2 * HALF, D), jnp.float32) for _ in range(NBUF)]
            + [pltpu.VMEM((LP,), jnp.int32),
               pltpu.VMEM((2, HALF), jnp.int32)]
            + [pltpu.SemaphoreType.DMA for _ in range(2 * NBUF + 1)]
        ),
    )(xf)


def kernel(x):
    batch_dims = x.shape[:-2]
    xf = x.reshape(B * L, D)
    return _imputer(xf).reshape(*batch_dims, L, D)
